# baseline (speedup 1.0000x reference)
from functools import partial

import jax
import jax.numpy as jnp
from jax.experimental import pallas as pl
from jax.experimental.pallas import tpu as pltpu


def _gx_kernel(x_ref, wih_ref, b_ref, gx_ref):
    xb = x_ref[...].astype(jnp.bfloat16)
    gx_ref[...] = (jnp.dot(xb, wih_ref[...], preferred_element_type=jnp.float32)
                   + b_ref[...]).astype(jnp.bfloat16)


def _rec_kernel(gx_ref, h0_ref, c0_ref, whh_ref,
                out_ref, hfin_ref, cfin_ref, h_scr, c_scr,
                *, C, n_chunks, B, H):
    k = pl.program_id(0)

    @pl.when(k == 0)
    def _():
        h_scr[...] = h0_ref[...]
        c_scr[...] = c0_ref[...]

    h_state = h_scr[...]
    c_state = c_scr[...]
    for c in range(C):
        hd = jnp.dot(h_state, whh_ref[...],
                     preferred_element_type=jnp.float32)
        gates = gx_ref[c * B:(c + 1) * B, :].astype(jnp.float32) + hd
        i_g = 0.5 * jnp.tanh(0.5 * gates[:, 0:H]) + 0.5
        f_g = 0.5 * jnp.tanh(0.5 * gates[:, H:2 * H]) + 0.5
        g_g = jnp.tanh(gates[:, 2 * H:3 * H])
        o_g = 0.5 * jnp.tanh(0.5 * gates[:, 3 * H:4 * H]) + 0.5
        c_state = f_g * c_state + i_g * g_g
        h_state = o_g * jnp.tanh(c_state)
        out_ref[c] = h_state
    h_scr[...] = h_state
    c_scr[...] = c_state

    @pl.when(k == n_chunks - 1)
    def _():
        hfin_ref[...] = h_state
        cfin_ref[...] = c_state


def kernel(x, h0, c0, W_ih, W_hh, b_ih, b_hh):
    T, B, I = x.shape
    H = h0.shape[1]
    C = 16
    n_chunks = T // C
    M = T * B

    wih_t = W_ih.T.astype(jnp.bfloat16)
    whh_t = W_hh.T
    b = (b_ih + b_hh).reshape(1, 4 * H)

    x2d = x.reshape(M, I)
    BM = min(1024, M)
    gx = pl.pallas_call(
        _gx_kernel,
        grid=(M // BM,),
        in_specs=[
            pl.BlockSpec((BM, I), lambda i: (i, 0)),
            pl.BlockSpec(memory_space=pltpu.VMEM),
            pl.BlockSpec(memory_space=pltpu.VMEM),
        ],
        out_specs=pl.BlockSpec((BM, 4 * H), lambda i: (i, 0)),
        out_shape=jax.ShapeDtypeStruct((M, 4 * H), jnp.bfloat16),
        compiler_params=pltpu.CompilerParams(
            dimension_semantics=("parallel",),
            vmem_limit_bytes=56 * 1024 * 1024,
        ),
        name="lstm_gx",
    )(x2d, wih_t, b)

    out_shape = (
        jax.ShapeDtypeStruct((T, B, H), jnp.float32),
        jax.ShapeDtypeStruct((B, H), jnp.float32),
        jax.ShapeDtypeStruct((B, H), jnp.float32),
    )
    outputs, h_fin, c_fin = pl.pallas_call(
        partial(_rec_kernel, C=C, n_chunks=n_chunks, B=B, H=H),
        grid=(n_chunks,),
        in_specs=[
            pl.BlockSpec((C * B, 4 * H), lambda k: (k, 0)),
            pl.BlockSpec(memory_space=pltpu.VMEM),
            pl.BlockSpec(memory_space=pltpu.VMEM),
            pl.BlockSpec(memory_space=pltpu.VMEM),
        ],
        out_specs=(
            pl.BlockSpec((C, B, H), lambda k: (k, 0, 0)),
            pl.BlockSpec((B, H), lambda k: (0, 0)),
            pl.BlockSpec((B, H), lambda k: (0, 0)),
        ),
        out_shape=out_shape,
        scratch_shapes=[
            pltpu.VMEM((B, H), jnp.float32),
            pltpu.VMEM((B, H), jnp.float32),
        ],
        compiler_params=pltpu.CompilerParams(
            dimension_semantics=("arbitrary",),
            vmem_limit_bytes=56 * 1024 * 1024,
        ),
        name="lstm_rec",
    )(gx, h0, c0, whh_t)
    return outputs, h_fin, c_fin


# speedup vs baseline: 1.0114x; 1.0114x over previous
from functools import partial

import jax
import jax.numpy as jnp
from jax.experimental import pallas as pl
from jax.experimental.pallas import tpu as pltpu


def _gx_kernel(x_ref, wih_ref, b_ref, gx_ref):
    xb = x_ref[...].astype(jnp.bfloat16)
    gx_ref[...] = (jnp.dot(xb, wih_ref[...], preferred_element_type=jnp.float32)
                   + b_ref[...]).astype(jnp.bfloat16)


def _rec_kernel(gx_ref, h0_ref, c0_ref, whh_ref,
                out_ref, hfin_ref, cfin_ref, h_scr, c_scr,
                *, C, n_chunks, B, H):
    k = pl.program_id(0)

    @pl.when(k == 0)
    def _():
        h_scr[...] = h0_ref[...]
        c_scr[...] = c0_ref[...]

    h_state = h_scr[...]
    c_state = c_scr[...]
    for c in range(C):
        hd = jnp.dot(h_state.astype(jnp.bfloat16), whh_ref[...],
                     preferred_element_type=jnp.float32)
        gates = gx_ref[c * B:(c + 1) * B, :].astype(jnp.float32) + hd
        i_g = 0.5 * jnp.tanh(0.5 * gates[:, 0:H]) + 0.5
        f_g = 0.5 * jnp.tanh(0.5 * gates[:, H:2 * H]) + 0.5
        g_g = jnp.tanh(gates[:, 2 * H:3 * H])
        o_g = 0.5 * jnp.tanh(0.5 * gates[:, 3 * H:4 * H]) + 0.5
        c_state = f_g * c_state + i_g * g_g
        h_state = o_g * jnp.tanh(c_state)
        out_ref[c] = h_state
    h_scr[...] = h_state
    c_scr[...] = c_state

    @pl.when(k == n_chunks - 1)
    def _():
        hfin_ref[...] = h_state
        cfin_ref[...] = c_state


def kernel(x, h0, c0, W_ih, W_hh, b_ih, b_hh):
    T, B, I = x.shape
    H = h0.shape[1]
    C = 16
    n_chunks = T // C
    M = T * B

    wih_t = W_ih.T.astype(jnp.bfloat16)
    whh_t = W_hh.T.astype(jnp.bfloat16)
    b = (b_ih + b_hh).reshape(1, 4 * H)

    x2d = x.reshape(M, I)
    BM = min(1024, M)
    gx = pl.pallas_call(
        _gx_kernel,
        grid=(M // BM,),
        in_specs=[
            pl.BlockSpec((BM, I), lambda i: (i, 0)),
            pl.BlockSpec(memory_space=pltpu.VMEM),
            pl.BlockSpec(memory_space=pltpu.VMEM),
        ],
        out_specs=pl.BlockSpec((BM, 4 * H), lambda i: (i, 0)),
        out_shape=jax.ShapeDtypeStruct((M, 4 * H), jnp.bfloat16),
        compiler_params=pltpu.CompilerParams(
            dimension_semantics=("parallel",),
            vmem_limit_bytes=56 * 1024 * 1024,
        ),
        name="lstm_gx",
    )(x2d, wih_t, b)

    out_shape = (
        jax.ShapeDtypeStruct((T, B, H), jnp.float32),
        jax.ShapeDtypeStruct((B, H), jnp.float32),
        jax.ShapeDtypeStruct((B, H), jnp.float32),
    )
    outputs, h_fin, c_fin = pl.pallas_call(
        partial(_rec_kernel, C=C, n_chunks=n_chunks, B=B, H=H),
        grid=(n_chunks,),
        in_specs=[
            pl.BlockSpec((C * B, 4 * H), lambda k: (k, 0)),
            pl.BlockSpec(memory_space=pltpu.VMEM),
            pl.BlockSpec(memory_space=pltpu.VMEM),
            pl.BlockSpec(memory_space=pltpu.VMEM),
        ],
        out_specs=(
            pl.BlockSpec((C, B, H), lambda k: (k, 0, 0)),
            pl.BlockSpec((B, H), lambda k: (0, 0)),
            pl.BlockSpec((B, H), lambda k: (0, 0)),
        ),
        out_shape=out_shape,
        scratch_shapes=[
            pltpu.VMEM((B, H), jnp.float32),
            pltpu.VMEM((B, H), jnp.float32),
        ],
        compiler_params=pltpu.CompilerParams(
            dimension_semantics=("arbitrary",),
            vmem_limit_bytes=56 * 1024 * 1024,
        ),
        name="lstm_rec",
    )(gx, h0, c0, whh_t)
    return outputs, h_fin, c_fin
